# baseline (speedup 1.0000x reference)
import jax
import jax.numpy as jnp
from jax import lax
from jax.experimental import pallas as pl
from jax.experimental.pallas import tpu as pltpu
from jax.experimental.pallas import tpu_sc as plsc

_NC = 2
_NS = 16
_NW = _NC * _NS
_L = 16

_ATTEN_COEFF = 0.2

_O_SIG, _O_TX = 0, 128
_N_IDX = 256
_N_FLT = 128

_TC_COLS = 256
_C_POW, _C_NODE0, _C_A, _C_B, _C_BW0, _N_BW = 0, 1, 33, 34, 64, 128


def _sc_body(R, CH, N):
    NCHUNK = R // _NW // CH
    NGRP = CH // _L
    RPW = R // _NW

    def body(x_hbm, idx_hbm, flt_hbm, out_hbm, bufa, bufb, idx_v, flt_v, oscr,
             idx_s, flt_s, sema, semb):
        cid = lax.axis_index("c")
        sid = lax.axis_index("s")
        wid = sid * _NC + cid
        pltpu.sync_copy(idx_hbm, idx_v)
        pltpu.sync_copy(flt_hbm, flt_v)

        def fill_idx(j, _):
            v = idx_v[pl.ds(j * _L, _L)]
            for i in range(_L):
                idx_s[j * _L + i] = v[i]
            return 0
        lax.fori_loop(0, _N_IDX // _L, fill_idx, 0)

        def fill_flt(j, _):
            v = flt_v[pl.ds(j * _L, _L)]
            for i in range(_L):
                flt_s[j * _L + i] = v[i]
            return 0
        lax.fori_loop(0, _N_FLT // _L, fill_flt, 0)

        row0 = wid * RPW
        lanes = lax.iota(jnp.int32, _L)
        zeros = jnp.zeros((_L,), jnp.float32)

        def start_copy(ci, buf, sem):
            pltpu.async_copy(x_hbm.at[pl.ds(row0 + ci * CH, CH)],
                             buf.at[:, pl.ds(0, N)], sem)

        def wait_copy(ci, buf, sem):
            pltpu.make_async_copy(x_hbm.at[pl.ds(row0 + ci * CH, CH)],
                                  buf.at[:, pl.ds(0, N)], sem).wait()

        def process(buf, ci, accs):
            def gat(rows, col):
                return plsc.load_gather(
                    buf, [rows, jnp.full((_L,), col, jnp.int32)])

            def group_body(g, a_att):
                rows = g * _L + lanes

                def tree8(g):
                    return (((g[0] + g[1]) + (g[2] + g[3]))
                            + ((g[4] + g[5]) + (g[6] + g[7])))

                def b_att(j, a):
                    t = [(gat(rows, idx_s[_O_SIG + j + k])
                          - gat(rows, idx_s[_O_TX + j + k])
                          + flt_s[j + k]) for k in range(8)]
                    sq = [x * x for x in t]
                    return a + tree8(sq)
                return plsc.parallel_loop(0, 128, step=8, carry=a_att)(b_att)

            return lax.fori_loop(0, NGRP, group_body, accs)

        start_copy(0, bufa, sema)
        start_copy(1, bufb, semb)

        def db_body(i, accs):
            ci0 = 2 * i
            wait_copy(ci0, bufa, sema)
            accs = process(bufa, ci0, accs)

            @pl.when(ci0 + 2 < NCHUNK)
            def _():
                start_copy(ci0 + 2, bufa, sema)
            wait_copy(ci0 + 1, bufb, semb)
            accs = process(bufb, ci0 + 1, accs)

            @pl.when(ci0 + 3 < NCHUNK)
            def _():
                start_copy(ci0 + 3, bufb, semb)
            return accs

        a_att = lax.fori_loop(0, NCHUNK // 2, db_body, zeros)
        oscr[0] = a_att
        pltpu.sync_copy(oscr, out_hbm.at[wid])

    return body


def _tc_body(nblk, blk, s_len):
    def body(x_ref, w_ref, cap_ref, out_ref, acc, carry):
        i = pl.program_id(0)

        @pl.when(i == 0)
        def _():
            acc[...] = jnp.zeros_like(acc)

        y = jnp.dot(x_ref[...], w_ref[...],
                    preferred_element_type=jnp.float32)
        cols = lax.broadcasted_iota(jnp.int32, (1, _TC_COLS), 1)
        sq_mask = ((cols >= _C_POW) & (cols < _C_NODE0 + 32)
                   ).astype(jnp.float32)

        ya = y[:, _C_A:_C_A + 1]
        yb = y[:, _C_B:_C_B + 1]
        shifted = jnp.concatenate([carry[...], yb[:-1, :]], axis=0)
        rid = i * blk + lax.broadcasted_iota(jnp.int32, (blk, 1), 0)
        first = (rid % s_len) == 0
        vio = ya + jnp.where(first, yb, shifted)
        carry[...] = yb[blk - 1:blk, :]

        exc = jnp.maximum(y[:, _C_BW0:_C_BW0 + _N_BW] - cap_ref[...], 0.0)

        hyd_oh = (cols == _C_A).astype(jnp.float32)
        cap_oh = (cols == _C_B).astype(jnp.float32)
        acc[...] += (jnp.sum(y * y, axis=0, keepdims=True) * sq_mask
                     + jnp.sum(vio * vio) * hyd_oh
                     + jnp.sum(exc * exc) * cap_oh)

        @pl.when(i == nblk - 1)
        def _():
            out_ref[...] = acc[...]

    return body


def kernel(predicted, targets, inflow_indices, outflow_indices, storage_indices,
           node_groups, generation_indices, load_indices, loss_indices,
           signal_indices, tx_power_indices, distance, bandwidth_indices, capacity):
    B, S, N = predicted.shape
    R = B * S
    CH = 64
    x = predicted.reshape(R, N)
    idx_all = jnp.concatenate([
        signal_indices, tx_power_indices,
    ]).astype(jnp.int32)
    flt_all = (_ATTEN_COEFF * distance).astype(jnp.float32)

    n_nodes, gsz = node_groups.shape
    n_gen, n_load, n_loss = (generation_indices.shape[0],
                             load_indices.shape[0], loss_indices.shape[0])
    n_in, n_out, n_st = (inflow_indices.shape[0], outflow_indices.shape[0],
                         storage_indices.shape[0])
    n_bw = bandwidth_indices.shape[0]
    w_rows = jnp.concatenate([
        generation_indices, load_indices, loss_indices,
        node_groups.reshape(-1),
        inflow_indices, outflow_indices, storage_indices,
        storage_indices,
        bandwidth_indices,
    ]).astype(jnp.int32)
    w_cols = jnp.concatenate([
        jnp.full((n_gen + n_load + n_loss,), _C_POW, jnp.int32),
        _C_NODE0 + jnp.repeat(jnp.arange(n_nodes, dtype=jnp.int32), gsz),
        jnp.full((n_in + n_out + n_st,), _C_A, jnp.int32),
        jnp.full((n_st,), _C_B, jnp.int32),
        _C_BW0 + jnp.arange(n_bw, dtype=jnp.int32),
    ])
    w_vals = jnp.concatenate([
        jnp.ones(n_gen, jnp.float32),
        -jnp.ones(n_load + n_loss, jnp.float32),
        jnp.ones(n_nodes * gsz, jnp.float32),
        jnp.ones(n_in, jnp.float32),
        -jnp.ones(n_out + n_st, jnp.float32),
        jnp.ones(n_st, jnp.float32),
        jnp.ones(n_bw, jnp.float32),
    ])
    row_oh = (jnp.arange(N, dtype=jnp.int32)[:, None]
              == w_rows[None, :]).astype(jnp.float32)
    col_oh = (w_cols[:, None]
              == jnp.arange(_TC_COLS, dtype=jnp.int32)[None, :]
              ).astype(jnp.float32) * w_vals[:, None]
    w_mat = jnp.dot(row_oh, col_oh, preferred_element_type=jnp.float32)

    mesh = plsc.VectorSubcoreMesh(core_axis_name="c", subcore_axis_name="s",
                                  num_cores=_NC, num_subcores=_NS)
    sc_out = pl.kernel(
        _sc_body(R, CH, N),
        out_type=jax.ShapeDtypeStruct((_NW, 1, _L), jnp.float32),
        mesh=mesh,
        compiler_params=pltpu.CompilerParams(use_tc_tiling_on_sc=False,
                                             needs_layout_passes=False),
        scratch_types=[
            pltpu.VMEM((CH, N + 8), jnp.float32),
            pltpu.VMEM((CH, N + 8), jnp.float32),
            pltpu.VMEM((_N_IDX,), jnp.int32),
            pltpu.VMEM((_N_FLT,), jnp.float32),
            pltpu.VMEM((1, _L), jnp.float32),
            pltpu.SMEM((_N_IDX,), jnp.int32),
            pltpu.SMEM((_N_FLT,), jnp.float32),
            pltpu.SemaphoreType.DMA,
            pltpu.SemaphoreType.DMA,
        ],
    )(x, idx_all, flt_all)

    BLK = 512
    nblk = R // BLK
    tc_out = pl.pallas_call(
        _tc_body(nblk, BLK, S),
        grid=(nblk,),
        in_specs=[
            pl.BlockSpec((BLK, N), lambda i: (i, 0)),
            pl.BlockSpec((N, _TC_COLS), lambda i: (0, 0)),
            pl.BlockSpec((1, _N_BW), lambda i: (0, 0)),
        ],
        out_specs=pl.BlockSpec((1, _TC_COLS), lambda i: (0, 0)),
        out_shape=jax.ShapeDtypeStruct((1, _TC_COLS), jnp.float32),
        scratch_shapes=[pltpu.VMEM((1, _TC_COLS), jnp.float32),
                        pltpu.VMEM((1, 1), jnp.float32)],
    )(x, w_mat, capacity.astype(jnp.float32).reshape(1, _N_BW))

    att_sum = sc_out.sum()
    n_att = signal_indices.shape[0]
    denom = jnp.float32(R)
    pow_sum = tc_out[0, _C_POW]
    kcl_sum = jnp.sum(tc_out[0, _C_NODE0:_C_NODE0 + n_nodes])
    hyd_sum = tc_out[0, _C_A]
    cap_sum = tc_out[0, _C_B]
    total = (hyd_sum / denom
             + kcl_sum / (denom * n_nodes)
             + pow_sum / denom
             + att_sum / (denom * n_att)
             + cap_sum / (denom * n_bw))
    return total.astype(jnp.float32)


# speedup vs baseline: 1.0261x; 1.0261x over previous
import jax
import jax.numpy as jnp
from jax import lax
from jax.experimental import pallas as pl
from jax.experimental.pallas import tpu as pltpu
from jax.experimental.pallas import tpu_sc as plsc

_NC = 2
_NS = 16
_NW = _NC * _NS
_L = 16

_ATTEN_COEFF = 0.2

_O_SIG, _O_TX, _O_BW = 0, 128, 256
_N_IDX = 384

_TC_COLS = 64
_C_POW, _C_NODE0, _C_A, _C_B = 0, 1, 33, 34


def _sc_body(R, CH, N):
    NCHUNK = R // _NW // CH
    NGRP = CH // _L
    RPW = R // _NW

    def body(x_hbm, idx_hbm, flt_hbm, out_hbm, bufa, bufb, idx_v, flt_v, oscr,
             idx_s, flt_s, sema, semb):
        cid = lax.axis_index("c")
        sid = lax.axis_index("s")
        wid = sid * _NC + cid
        pltpu.sync_copy(idx_hbm, idx_v)
        pltpu.sync_copy(flt_hbm, flt_v)

        def fill_idx(j, _):
            v = idx_v[pl.ds(j * _L, _L)]
            for i in range(_L):
                idx_s[j * _L + i] = v[i]
            return 0
        lax.fori_loop(0, _N_IDX // _L, fill_idx, 0)

        def fill_flt(j, _):
            v = flt_v[pl.ds(j * _L, _L)]
            for i in range(_L):
                flt_s[j * _L + i] = v[i]
            return 0
        lax.fori_loop(0, 256 // _L, fill_flt, 0)

        row0 = wid * RPW
        lanes = lax.iota(jnp.int32, _L)
        zeros = jnp.zeros((_L,), jnp.float32)

        def start_copy(ci, buf, sem):
            pltpu.async_copy(x_hbm.at[pl.ds(row0 + ci * CH, CH)],
                             buf.at[:, pl.ds(0, N)], sem)

        def wait_copy(ci, buf, sem):
            pltpu.make_async_copy(x_hbm.at[pl.ds(row0 + ci * CH, CH)],
                                  buf.at[:, pl.ds(0, N)], sem).wait()

        def process(buf, ci, accs):
            def gat(rows, col):
                return plsc.load_gather(
                    buf, [rows, jnp.full((_L,), col, jnp.int32)])

            def group_body(g, accs):
                a_att, a_cap = accs
                rows = g * _L + lanes

                def tree8(g):
                    return (((g[0] + g[1]) + (g[2] + g[3]))
                            + ((g[4] + g[5]) + (g[6] + g[7])))

                def b_att(j, a):
                    t = [(gat(rows, idx_s[_O_SIG + j + k])
                          - gat(rows, idx_s[_O_TX + j + k])
                          + flt_s[j + k]) for k in range(8)]
                    sq = [x * x for x in t]
                    return a + tree8(sq)
                a_att = plsc.parallel_loop(0, 128, step=8, carry=a_att)(b_att)

                def b_cap(j, a):
                    e = [jnp.maximum(gat(rows, idx_s[_O_BW + j + k])
                                     - flt_s[128 + j + k], 0.0)
                         for k in range(8)]
                    sq = [x * x for x in e]
                    return a + tree8(sq)
                a_cap = plsc.parallel_loop(0, 128, step=8, carry=a_cap)(b_cap)

                return a_att, a_cap

            return lax.fori_loop(0, NGRP, group_body, accs)

        start_copy(0, bufa, sema)
        start_copy(1, bufb, semb)

        def db_body(i, accs):
            ci0 = 2 * i
            wait_copy(ci0, bufa, sema)
            accs = process(bufa, ci0, accs)

            @pl.when(ci0 + 2 < NCHUNK)
            def _():
                start_copy(ci0 + 2, bufa, sema)
            wait_copy(ci0 + 1, bufb, semb)
            accs = process(bufb, ci0 + 1, accs)

            @pl.when(ci0 + 3 < NCHUNK)
            def _():
                start_copy(ci0 + 3, bufb, semb)
            return accs

        accs = lax.fori_loop(0, NCHUNK // 2, db_body, (zeros, zeros))
        for i in range(2):
            oscr[i] = accs[i]
        pltpu.sync_copy(oscr, out_hbm.at[wid])

    return body


def _tc_body(nblk, blk, s_len):
    def body(x_ref, w_ref, out_ref, acc, carry):
        i = pl.program_id(0)

        @pl.when(i == 0)
        def _():
            acc[...] = jnp.zeros_like(acc)

        y = jnp.dot(x_ref[...], w_ref[...],
                    preferred_element_type=jnp.float32)
        cols = lax.broadcasted_iota(jnp.int32, (1, _TC_COLS), 1)
        sq_mask = ((cols >= _C_POW) & (cols < _C_NODE0 + 32)
                   ).astype(jnp.float32)

        ya = y[:, _C_A:_C_A + 1]
        yb = y[:, _C_B:_C_B + 1]
        shifted = jnp.concatenate([carry[...], yb[:-1, :]], axis=0)
        rid = i * blk + lax.broadcasted_iota(jnp.int32, (blk, 1), 0)
        first = (rid % s_len) == 0
        vio = ya + jnp.where(first, yb, shifted)
        carry[...] = yb[blk - 1:blk, :]

        hyd_oh = (cols == _C_A).astype(jnp.float32)
        acc[...] += (jnp.sum(y * y, axis=0, keepdims=True) * sq_mask
                     + jnp.sum(vio * vio) * hyd_oh)

        @pl.when(i == nblk - 1)
        def _():
            out_ref[...] = acc[...]

    return body


def kernel(predicted, targets, inflow_indices, outflow_indices, storage_indices,
           node_groups, generation_indices, load_indices, loss_indices,
           signal_indices, tx_power_indices, distance, bandwidth_indices, capacity):
    B, S, N = predicted.shape
    R = B * S
    CH = 64
    x = predicted.reshape(R, N)
    idx_all = jnp.concatenate([
        signal_indices, tx_power_indices, bandwidth_indices,
    ]).astype(jnp.int32)
    flt_all = jnp.concatenate([_ATTEN_COEFF * distance, capacity]
                              ).astype(jnp.float32)

    n_nodes, gsz = node_groups.shape
    n_gen, n_load, n_loss = (generation_indices.shape[0],
                             load_indices.shape[0], loss_indices.shape[0])
    n_in, n_out, n_st = (inflow_indices.shape[0], outflow_indices.shape[0],
                         storage_indices.shape[0])
    w_rows = jnp.concatenate([
        generation_indices, load_indices, loss_indices,
        node_groups.reshape(-1),
        inflow_indices, outflow_indices, storage_indices,
        storage_indices,
    ]).astype(jnp.int32)
    w_cols = jnp.concatenate([
        jnp.full((n_gen + n_load + n_loss,), _C_POW, jnp.int32),
        _C_NODE0 + jnp.repeat(jnp.arange(n_nodes, dtype=jnp.int32), gsz),
        jnp.full((n_in + n_out + n_st,), _C_A, jnp.int32),
        jnp.full((n_st,), _C_B, jnp.int32),
    ])
    w_vals = jnp.concatenate([
        jnp.ones(n_gen, jnp.float32),
        -jnp.ones(n_load + n_loss, jnp.float32),
        jnp.ones(n_nodes * gsz, jnp.float32),
        jnp.ones(n_in, jnp.float32),
        -jnp.ones(n_out + n_st, jnp.float32),
        jnp.ones(n_st, jnp.float32),
    ])
    row_oh = (jnp.arange(N, dtype=jnp.int32)[:, None]
              == w_rows[None, :]).astype(jnp.float32)
    col_oh = (w_cols[:, None]
              == jnp.arange(_TC_COLS, dtype=jnp.int32)[None, :]
              ).astype(jnp.float32) * w_vals[:, None]
    w_mat = jnp.dot(row_oh, col_oh, preferred_element_type=jnp.float32)

    mesh = plsc.VectorSubcoreMesh(core_axis_name="c", subcore_axis_name="s",
                                  num_cores=_NC, num_subcores=_NS)
    sc_out = pl.kernel(
        _sc_body(R, CH, N),
        out_type=jax.ShapeDtypeStruct((_NW, 2, _L), jnp.float32),
        mesh=mesh,
        compiler_params=pltpu.CompilerParams(use_tc_tiling_on_sc=False,
                                             needs_layout_passes=False),
        scratch_types=[
            pltpu.VMEM((CH, N + 8), jnp.float32),
            pltpu.VMEM((CH, N + 8), jnp.float32),
            pltpu.VMEM((_N_IDX,), jnp.int32),
            pltpu.VMEM((256,), jnp.float32),
            pltpu.VMEM((2, _L), jnp.float32),
            pltpu.SMEM((_N_IDX,), jnp.int32),
            pltpu.SMEM((256,), jnp.float32),
            pltpu.SemaphoreType.DMA,
            pltpu.SemaphoreType.DMA,
        ],
    )(x, idx_all, flt_all)

    BLK = 512
    nblk = R // BLK
    tc_out = pl.pallas_call(
        _tc_body(nblk, BLK, S),
        grid=(nblk,),
        in_specs=[
            pl.BlockSpec((BLK, N), lambda i: (i, 0)),
            pl.BlockSpec((N, _TC_COLS), lambda i: (0, 0)),
        ],
        out_specs=pl.BlockSpec((1, _TC_COLS), lambda i: (0, 0)),
        out_shape=jax.ShapeDtypeStruct((1, _TC_COLS), jnp.float32),
        scratch_shapes=[pltpu.VMEM((1, _TC_COLS), jnp.float32),
                        pltpu.VMEM((1, 1), jnp.float32)],
    )(x, w_mat)

    part = sc_out.sum(axis=(0, 2))
    n_att = signal_indices.shape[0]
    n_cap = bandwidth_indices.shape[0]
    denom = jnp.float32(R)
    pow_sum = tc_out[0, _C_POW]
    kcl_sum = jnp.sum(tc_out[0, _C_NODE0:_C_NODE0 + n_nodes])
    hyd_sum = tc_out[0, _C_A]
    total = (hyd_sum / denom
             + kcl_sum / (denom * n_nodes)
             + pow_sum / denom
             + part[0] / (denom * n_att)
             + part[1] / (denom * n_cap))
    return total.astype(jnp.float32)
